# baseline (device time: 12695 ns/iter reference)
import jax
import jax.numpy as jnp
from jax import lax
from jax.experimental import pallas as pl
from jax.experimental.pallas import tpu as pltpu

FWD_CHUNK = 2


def kernel(A, B):
    m, k = A.shape
    _, n = B.shape
    nh = n // 2
    kc = k // FWD_CHUNK

    def body(a_ref, b_ref, out_ref, a16_ref, bhalf_ref, a_rx, bdir_rx,
             bfwd_rx, send_sems, a_recv_sem, bdir_sems, bfwd_sems):
        my_x = lax.axis_index("x")
        my_y = lax.axis_index("y")
        peer = (my_x, 1 - my_y)
        xnbr = (1 - my_x, my_y)

        barrier_sem = pltpu.get_barrier_semaphore()
        for nbr in (peer, xnbr):
            pl.semaphore_signal(
                barrier_sem, inc=1, device_id=nbr,
                device_id_type=pl.DeviceIdType.MESH,
            )

        b16 = b_ref[...].astype(jnp.bfloat16)
        bhalf_ref[...] = jnp.where(my_x == 0, b16[:, :nh], b16[:, nh:])
        a16_ref[...] = a_ref[...].astype(jnp.bfloat16)

        pl.semaphore_wait(barrier_sem, 2)

        sends = []
        for c in range(FWD_CHUNK):
            rows = pl.ds(c * kc, kc)
            s = pltpu.make_async_remote_copy(
                src_ref=bhalf_ref.at[rows, :],
                dst_ref=bdir_rx.at[rows, :],
                send_sem=send_sems.at[c],
                recv_sem=bdir_sems.at[c],
                device_id=peer,
                device_id_type=pl.DeviceIdType.MESH,
            )
            s.start()
            sends.append(s)
        a_send = pltpu.make_async_remote_copy(
            src_ref=a16_ref,
            dst_ref=a_rx,
            send_sem=send_sems.at[FWD_CHUNK],
            recv_sem=a_recv_sem,
            device_id=peer,
            device_id_type=pl.DeviceIdType.MESH,
        )
        a_send.start()
        sends.append(a_send)

        out_ref[...] = jnp.dot(a16_ref[...], b16,
                               preferred_element_type=jnp.float32)

        fwds = []
        for c in range(FWD_CHUNK):
            rows = pl.ds(c * kc, kc)
            pltpu.make_async_remote_copy(
                src_ref=bhalf_ref.at[rows, :], dst_ref=bdir_rx.at[rows, :],
                send_sem=send_sems.at[c], recv_sem=bdir_sems.at[c],
                device_id=peer, device_id_type=pl.DeviceIdType.MESH,
            ).wait_recv()
            f = pltpu.make_async_remote_copy(
                src_ref=bdir_rx.at[rows, :],
                dst_ref=bfwd_rx.at[rows, :],
                send_sem=send_sems.at[FWD_CHUNK + 1 + c],
                recv_sem=bfwd_sems.at[c],
                device_id=xnbr,
                device_id_type=pl.DeviceIdType.MESH,
            )
            f.start()
            fwds.append(f)

        a_send_dummy = a_send
        a_send_dummy.wait_recv()
        for f in fwds:
            f.wait_recv()

        bdir = bdir_rx[...]
        bfwd = bfwd_rx[...]
        b_left = jnp.where(my_x == 0, bdir, bfwd)
        b_right = jnp.where(my_x == 0, bfwd, bdir)
        a_peer = a_rx[...]
        out_ref[:, :nh] = out_ref[:, :nh] + jnp.dot(
            a_peer, b_left, preferred_element_type=jnp.float32)
        out_ref[:, nh:] = out_ref[:, nh:] + jnp.dot(
            a_peer, b_right, preferred_element_type=jnp.float32)

        for s in sends:
            s.wait_send()
        for f in fwds:
            f.wait_send()

    return pl.pallas_call(
        body,
        out_shape=jax.ShapeDtypeStruct((m, n), jnp.float32),
        in_specs=[
            pl.BlockSpec(memory_space=pltpu.VMEM),
            pl.BlockSpec(memory_space=pltpu.VMEM),
        ],
        out_specs=pl.BlockSpec(memory_space=pltpu.VMEM),
        scratch_shapes=[
            pltpu.VMEM((m, k), jnp.bfloat16),
            pltpu.VMEM((k, n // 2), jnp.bfloat16),
            pltpu.VMEM((m, k), jnp.bfloat16),
            pltpu.VMEM((k, n // 2), jnp.bfloat16),
            pltpu.VMEM((k, n // 2), jnp.bfloat16),
            pltpu.SemaphoreType.DMA((2 * FWD_CHUNK + 1,)),
            pltpu.SemaphoreType.DMA,
            pltpu.SemaphoreType.DMA((FWD_CHUNK,)),
            pltpu.SemaphoreType.DMA((FWD_CHUNK,)),
        ],
        compiler_params=pltpu.CompilerParams(collective_id=0),
    )(A, B)


# device time: 12577 ns/iter; 1.0094x vs baseline; 1.0094x over previous
import jax
import jax.numpy as jnp
from jax import lax
from jax.experimental import pallas as pl
from jax.experimental.pallas import tpu as pltpu

FWD_CHUNK = 4


def kernel(A, B):
    m, k = A.shape
    _, n = B.shape
    nh = n // 2
    kc = k // FWD_CHUNK

    def body(a_ref, b_ref, out_ref, a16_ref, bhalf_ref, a_rx, bdir_rx,
             bfwd_rx, send_sems, a_recv_sem, bdir_sems, bfwd_sems):
        my_x = lax.axis_index("x")
        my_y = lax.axis_index("y")
        peer = (my_x, 1 - my_y)
        xnbr = (1 - my_x, my_y)

        barrier_sem = pltpu.get_barrier_semaphore()
        for nbr in (peer, xnbr):
            pl.semaphore_signal(
                barrier_sem, inc=1, device_id=nbr,
                device_id_type=pl.DeviceIdType.MESH,
            )

        b16 = b_ref[...].astype(jnp.bfloat16)
        bhalf_ref[...] = jnp.where(my_x == 0, b16[:, :nh], b16[:, nh:])
        a16_ref[...] = a_ref[...].astype(jnp.bfloat16)

        pl.semaphore_wait(barrier_sem, 2)

        sends = []
        for c in range(FWD_CHUNK):
            rows = pl.ds(c * kc, kc)
            s = pltpu.make_async_remote_copy(
                src_ref=bhalf_ref.at[rows, :],
                dst_ref=bdir_rx.at[rows, :],
                send_sem=send_sems.at[c],
                recv_sem=bdir_sems.at[c],
                device_id=peer,
                device_id_type=pl.DeviceIdType.MESH,
            )
            s.start()
            sends.append(s)
        a_send = pltpu.make_async_remote_copy(
            src_ref=a16_ref,
            dst_ref=a_rx,
            send_sem=send_sems.at[FWD_CHUNK],
            recv_sem=a_recv_sem,
            device_id=peer,
            device_id_type=pl.DeviceIdType.MESH,
        )
        a_send.start()
        sends.append(a_send)

        fwds = []
        for c in range(FWD_CHUNK):
            rows = pl.ds(c * kc, kc)
            sends[c].wait_recv()
            f = pltpu.make_async_remote_copy(
                src_ref=bdir_rx.at[rows, :],
                dst_ref=bfwd_rx.at[rows, :],
                send_sem=send_sems.at[FWD_CHUNK + 1 + c],
                recv_sem=bfwd_sems.at[c],
                device_id=xnbr,
                device_id_type=pl.DeviceIdType.MESH,
            )
            f.start()
            fwds.append(f)

        local = jnp.dot(a16_ref[...], b16, preferred_element_type=jnp.float32)

        a_send.wait_recv()
        for f in fwds:
            f.wait_recv()

        bdir = bdir_rx[...]
        bfwd = bfwd_rx[...]
        b_left = jnp.where(my_x == 0, bdir, bfwd)
        b_right = jnp.where(my_x == 0, bfwd, bdir)
        a_peer = a_rx[...]
        out_ref[:, :nh] = (
            local[:, :nh]
            + jnp.dot(a_peer, b_left, preferred_element_type=jnp.float32)
        ).astype(jnp.bfloat16)
        out_ref[:, nh:] = (
            local[:, nh:]
            + jnp.dot(a_peer, b_right, preferred_element_type=jnp.float32)
        ).astype(jnp.bfloat16)

        for s in sends:
            s.wait_send()
        for f in fwds:
            f.wait_send()

    return pl.pallas_call(
        body,
        out_shape=jax.ShapeDtypeStruct((m, n), jnp.bfloat16),
        in_specs=[
            pl.BlockSpec(memory_space=pltpu.VMEM),
            pl.BlockSpec(memory_space=pltpu.VMEM),
        ],
        out_specs=pl.BlockSpec(memory_space=pltpu.VMEM),
        scratch_shapes=[
            pltpu.VMEM((m, k), jnp.bfloat16),
            pltpu.VMEM((k, n // 2), jnp.bfloat16),
            pltpu.VMEM((m, k), jnp.bfloat16),
            pltpu.VMEM((k, n // 2), jnp.bfloat16),
            pltpu.VMEM((k, n // 2), jnp.bfloat16),
            pltpu.SemaphoreType.DMA((2 * FWD_CHUNK + 1,)),
            pltpu.SemaphoreType.DMA,
            pltpu.SemaphoreType.DMA((FWD_CHUNK,)),
            pltpu.SemaphoreType.DMA((FWD_CHUNK,)),
        ],
        compiler_params=pltpu.CompilerParams(collective_id=0),
    )(A, B)


# device time: 12513 ns/iter; 1.0145x vs baseline; 1.0051x over previous
import jax
import jax.numpy as jnp
from jax import lax
from jax.experimental import pallas as pl
from jax.experimental.pallas import tpu as pltpu

N_BCHUNK = 4
N_ACHUNK = 2


def kernel(A, B):
    m, k = A.shape
    _, n = B.shape
    nh = n // 2
    kb = k // N_BCHUNK
    ka = k // N_ACHUNK

    def body(a_ref, b_ref, out_ref, a16_ref, bhalf_ref, a_rx, bdir_rx,
             bfwd_rx, send_sems, a_recv_sems, bdir_sems, bfwd_sems):
        my_x = lax.axis_index("x")
        my_y = lax.axis_index("y")
        peer = (my_x, 1 - my_y)
        xnbr = (1 - my_x, my_y)

        barrier_sem = pltpu.get_barrier_semaphore()
        for nbr in (peer, xnbr):
            pl.semaphore_signal(
                barrier_sem, inc=1, device_id=nbr,
                device_id_type=pl.DeviceIdType.MESH,
            )

        b16 = b_ref[...].astype(jnp.bfloat16)
        bhalf_ref[...] = jnp.where(my_x == 0, b16[:, :nh], b16[:, nh:])

        pl.semaphore_wait(barrier_sem, 2)

        sends = []
        for c in range(N_BCHUNK):
            rows = pl.ds(c * kb, kb)
            s = pltpu.make_async_remote_copy(
                src_ref=bhalf_ref.at[rows, :],
                dst_ref=bdir_rx.at[rows, :],
                send_sem=send_sems.at[c],
                recv_sem=bdir_sems.at[c],
                device_id=peer,
                device_id_type=pl.DeviceIdType.MESH,
            )
            s.start()
            sends.append(s)

        a16_ref[...] = a_ref[...].astype(jnp.bfloat16)
        a_sends = []
        for c in range(N_ACHUNK):
            cols = pl.ds(c * ka, ka)
            s = pltpu.make_async_remote_copy(
                src_ref=a16_ref.at[:, cols],
                dst_ref=a_rx.at[:, cols],
                send_sem=send_sems.at[N_BCHUNK + c],
                recv_sem=a_recv_sems.at[c],
                device_id=peer,
                device_id_type=pl.DeviceIdType.MESH,
            )
            s.start()
            a_sends.append(s)

        fwds = []
        for c in range(N_BCHUNK):
            rows = pl.ds(c * kb, kb)
            sends[c].wait_recv()
            f = pltpu.make_async_remote_copy(
                src_ref=bdir_rx.at[rows, :],
                dst_ref=bfwd_rx.at[rows, :],
                send_sem=send_sems.at[N_BCHUNK + N_ACHUNK + c],
                recv_sem=bfwd_sems.at[c],
                device_id=xnbr,
                device_id_type=pl.DeviceIdType.MESH,
            )
            f.start()
            fwds.append(f)

        acc_l = jnp.dot(a16_ref[...], b16[:, :nh],
                        preferred_element_type=jnp.float32)
        acc_r = jnp.dot(a16_ref[...], b16[:, nh:],
                        preferred_element_type=jnp.float32)

        rel = ka // kb
        for c in range(N_ACHUNK):
            a_sends[c].wait_recv()
            for r in range(c * rel, (c + 1) * rel):
                fwds[r].wait_recv()
            a_c = a_rx[:, c * ka:(c + 1) * ka]
            bdir_c = bdir_rx[c * ka:(c + 1) * ka, :]
            bfwd_c = bfwd_rx[c * ka:(c + 1) * ka, :]
            bl_c = jnp.where(my_x == 0, bdir_c, bfwd_c)
            br_c = jnp.where(my_x == 0, bfwd_c, bdir_c)
            acc_l = acc_l + jnp.dot(a_c, bl_c,
                                    preferred_element_type=jnp.float32)
            acc_r = acc_r + jnp.dot(a_c, br_c,
                                    preferred_element_type=jnp.float32)

        out_ref[:, :nh] = acc_l.astype(jnp.bfloat16)
        out_ref[:, nh:] = acc_r.astype(jnp.bfloat16)

        for s in sends + a_sends + fwds:
            s.wait_send()

    return pl.pallas_call(
        body,
        out_shape=jax.ShapeDtypeStruct((m, n), jnp.bfloat16),
        in_specs=[
            pl.BlockSpec(memory_space=pltpu.VMEM),
            pl.BlockSpec(memory_space=pltpu.VMEM),
        ],
        out_specs=pl.BlockSpec(memory_space=pltpu.VMEM),
        scratch_shapes=[
            pltpu.VMEM((m, k), jnp.bfloat16),
            pltpu.VMEM((k, n // 2), jnp.bfloat16),
            pltpu.VMEM((m, k), jnp.bfloat16),
            pltpu.VMEM((k, n // 2), jnp.bfloat16),
            pltpu.VMEM((k, n // 2), jnp.bfloat16),
            pltpu.SemaphoreType.DMA((2 * N_BCHUNK + N_ACHUNK,)),
            pltpu.SemaphoreType.DMA((N_ACHUNK,)),
            pltpu.SemaphoreType.DMA((N_BCHUNK,)),
            pltpu.SemaphoreType.DMA((N_BCHUNK,)),
        ],
        compiler_params=pltpu.CompilerParams(collective_id=0),
    )(A, B)


# device time: 10380 ns/iter; 1.2230x vs baseline; 1.2055x over previous
import jax
import jax.numpy as jnp
from jax import lax
from jax.experimental import pallas as pl
from jax.experimental.pallas import tpu as pltpu

N_BCHUNK = 4
N_ACHUNK = 2


def kernel(A, B):
    m, k = A.shape
    _, n = B.shape
    nh = n // 2
    kb = k // N_BCHUNK
    ka = k // N_ACHUNK

    def body(a_hbm, b_hbm, out_ref, a_ref, b_ref, a16_ref, bhalf_ref, a_rx,
             bdir_rx, bfwd_rx, load_sems, send_sems, a_recv_sems, bdir_sems,
             bfwd_sems):
        my_x = lax.axis_index("x")
        my_y = lax.axis_index("y")
        peer = (my_x, 1 - my_y)
        xnbr = (1 - my_x, my_y)

        barrier_sem = pltpu.get_barrier_semaphore()
        for nbr in (peer, xnbr):
            pl.semaphore_signal(
                barrier_sem, inc=1, device_id=nbr,
                device_id_type=pl.DeviceIdType.MESH,
            )

        b_cp = pltpu.make_async_copy(b_hbm, b_ref, load_sems.at[1])
        a_cp = pltpu.make_async_copy(a_hbm, a_ref, load_sems.at[0])
        b_cp.start()
        a_cp.start()

        b_cp.wait()
        b16 = b_ref[...].astype(jnp.bfloat16)
        bhalf_ref[...] = jnp.where(my_x == 0, b16[:, :nh], b16[:, nh:])

        pl.semaphore_wait(barrier_sem, 2)

        sends = []
        for c in range(N_BCHUNK):
            rows = pl.ds(c * kb, kb)
            s = pltpu.make_async_remote_copy(
                src_ref=bhalf_ref.at[rows, :],
                dst_ref=bdir_rx.at[rows, :],
                send_sem=send_sems.at[c],
                recv_sem=bdir_sems.at[c],
                device_id=peer,
                device_id_type=pl.DeviceIdType.MESH,
            )
            s.start()
            sends.append(s)

        a_cp.wait()
        a16_ref[...] = a_ref[...].astype(jnp.bfloat16)
        a_sends = []
        for c in range(N_ACHUNK):
            cols = pl.ds(c * ka, ka)
            s = pltpu.make_async_remote_copy(
                src_ref=a16_ref.at[:, cols],
                dst_ref=a_rx.at[:, cols],
                send_sem=send_sems.at[N_BCHUNK + c],
                recv_sem=a_recv_sems.at[c],
                device_id=peer,
                device_id_type=pl.DeviceIdType.MESH,
            )
            s.start()
            a_sends.append(s)

        fwds = []
        for c in range(N_BCHUNK):
            rows = pl.ds(c * kb, kb)
            sends[c].wait_recv()
            f = pltpu.make_async_remote_copy(
                src_ref=bdir_rx.at[rows, :],
                dst_ref=bfwd_rx.at[rows, :],
                send_sem=send_sems.at[N_BCHUNK + N_ACHUNK + c],
                recv_sem=bfwd_sems.at[c],
                device_id=xnbr,
                device_id_type=pl.DeviceIdType.MESH,
            )
            f.start()
            fwds.append(f)

        acc_l = jnp.dot(a16_ref[...], b16[:, :nh],
                        preferred_element_type=jnp.float32)
        acc_r = jnp.dot(a16_ref[...], b16[:, nh:],
                        preferred_element_type=jnp.float32)

        rel = ka // kb
        for c in range(N_ACHUNK):
            a_sends[c].wait_recv()
            for r in range(c * rel, (c + 1) * rel):
                fwds[r].wait_recv()
            a_c = a_rx[:, c * ka:(c + 1) * ka]
            bdir_c = bdir_rx[c * ka:(c + 1) * ka, :]
            bfwd_c = bfwd_rx[c * ka:(c + 1) * ka, :]
            bl_c = jnp.where(my_x == 0, bdir_c, bfwd_c)
            br_c = jnp.where(my_x == 0, bfwd_c, bdir_c)
            acc_l = acc_l + jnp.dot(a_c, bl_c,
                                    preferred_element_type=jnp.float32)
            acc_r = acc_r + jnp.dot(a_c, br_c,
                                    preferred_element_type=jnp.float32)

        out_ref[:, :nh] = acc_l.astype(jnp.bfloat16)
        out_ref[:, nh:] = acc_r.astype(jnp.bfloat16)

        for s in sends + a_sends + fwds:
            s.wait_send()

    return pl.pallas_call(
        body,
        out_shape=jax.ShapeDtypeStruct((m, n), jnp.bfloat16),
        in_specs=[
            pl.BlockSpec(memory_space=pltpu.MemorySpace.HBM),
            pl.BlockSpec(memory_space=pltpu.MemorySpace.HBM),
        ],
        out_specs=pl.BlockSpec(memory_space=pltpu.VMEM),
        scratch_shapes=[
            pltpu.VMEM((m, k), jnp.float32),
            pltpu.VMEM((k, n), jnp.float32),
            pltpu.VMEM((m, k), jnp.bfloat16),
            pltpu.VMEM((k, n // 2), jnp.bfloat16),
            pltpu.VMEM((m, k), jnp.bfloat16),
            pltpu.VMEM((k, n // 2), jnp.bfloat16),
            pltpu.VMEM((k, n // 2), jnp.bfloat16),
            pltpu.SemaphoreType.DMA((2,)),
            pltpu.SemaphoreType.DMA((2 * N_BCHUNK + N_ACHUNK,)),
            pltpu.SemaphoreType.DMA((N_ACHUNK,)),
            pltpu.SemaphoreType.DMA((N_BCHUNK,)),
            pltpu.SemaphoreType.DMA((N_BCHUNK,)),
        ],
        compiler_params=pltpu.CompilerParams(collective_id=0),
    )(
        pltpu.with_memory_space_constraint(A, pltpu.MemorySpace.HBM),
        pltpu.with_memory_space_constraint(B, pltpu.MemorySpace.HBM),
    )
